# baseline (speedup 1.0000x reference)
import functools

import jax
import jax.numpy as jnp
from jax import lax
from jax.experimental import pallas as pl
from jax.experimental.pallas import tpu as pltpu
from jax.experimental.pallas import tpu_sc as plsc

B, T, D = 4, 2048, 1024
NC, NS = 2, 16
NW = NC * NS
TPW = T // NW
CH = 8
NCH = TPW // CH
NITEMS = NCH * B
NBUF = 4
VL = 16
UNROLL = 8

_mesh = plsc.VectorSubcoreMesh(core_axis_name="c", subcore_axis_name="s")


@functools.partial(
    pl.kernel,
    out_type=jax.ShapeDtypeStruct((B * T * D,), jnp.float32),
    mesh=_mesh,
    scratch_types=[
        pltpu.VMEM((TPW * D,), jnp.float32),
        pltpu.VMEM((NBUF, CH * D), jnp.float32),
        pltpu.SemaphoreType.DMA,
        pltpu.SemaphoreType.DMA,
        pltpu.SemaphoreType.DMA((NBUF,)),
        pltpu.SemaphoreType.DMA((NBUF,)),
    ],
)
def _sc_add(tok_hbm, pos_hbm, out_hbm, pbuf, abuf, psem0, psem1, isem, osem):
    wid = lax.axis_index("s") * NC + lax.axis_index("c")
    t0 = wid * TPW

    pos_cp0 = pltpu.async_copy(
        pos_hbm.at[pl.ds(t0 * D, CH * D)], pbuf.at[pl.ds(0, CH * D)], psem0)
    pos_cp1 = pltpu.async_copy(
        pos_hbm.at[pl.ds((t0 + CH) * D, (TPW - CH) * D)],
        pbuf.at[pl.ds(CH * D, (TPW - CH) * D)], psem1)

    def tok_off(item):
        cc, b = divmod(item, B)
        return b * (T * D) + (t0 + cc * CH) * D

    def start_in(item):
        buf = item % NBUF
        return pltpu.async_copy(
            tok_hbm.at[pl.ds(tok_off(item), CH * D)], abuf.at[buf],
            isem.at[buf])

    in_cp = [None] * NITEMS
    out_cp = [None] * NITEMS
    in_cp[0] = start_in(0)
    in_cp[1] = start_in(1)

    for g in range(NITEMS):
        if g + 2 < NITEMS:
            if g - 2 >= 0:
                out_cp[g - 2].wait()
            in_cp[g + 2] = start_in(g + 2)

        cc = g // B
        if g == 0:
            pos_cp0.wait()
        elif g == B:
            pos_cp1.wait()

        buf = g % NBUF
        in_cp[g].wait()
        ab = abuf.at[buf]
        pbase = cc * CH * D

        def body(i, _, ab=ab, pbase=pbase):
            base = i * (VL * UNROLL)
            for k in range(UNROLL):
                off = base + k * VL
                plsc.addupdate(ab.at[pl.ds(off, VL)],
                               pbuf[pl.ds(pbase + off, VL)])
            return 0

        lax.fori_loop(0, CH * D // (VL * UNROLL), body, 0)

        out_cp[g] = pltpu.async_copy(
            ab, out_hbm.at[pl.ds(tok_off(g), CH * D)], osem.at[buf])

    out_cp[NITEMS - 2].wait()
    out_cp[NITEMS - 1].wait()


def kernel(encoded_tokens, pos_table):
    tok_flat = encoded_tokens.reshape(B * T * D)
    pos_flat = pos_table.reshape(T * D)
    out_flat = _sc_add(tok_flat, pos_flat)
    return out_flat.reshape(B, T, D)


# speedup vs baseline: 1.2775x; 1.2775x over previous
import functools

import jax
import jax.numpy as jnp
from jax import lax
from jax.experimental import pallas as pl
from jax.experimental.pallas import tpu as pltpu
from jax.experimental.pallas import tpu_sc as plsc

B, T, D = 4, 2048, 1024
NC, NS = 2, 16
NW = NC * NS
TPW = T // NW
CH = 8
NCH = TPW // CH
NITEMS = NCH * B
NBUF = 4
VL = 16
UNROLL = 8

_mesh = plsc.VectorSubcoreMesh(core_axis_name="c", subcore_axis_name="s")


@functools.partial(
    pl.kernel,
    out_type=jax.ShapeDtypeStruct((B * T * D,), jnp.float32),
    mesh=_mesh,
    scratch_types=[
        pltpu.VMEM((TPW * D,), jnp.float32),
        pltpu.VMEM((NBUF, CH * D), jnp.float32),
        pltpu.SemaphoreType.DMA,
        pltpu.SemaphoreType.DMA,
        pltpu.SemaphoreType.DMA((NBUF,)),
        pltpu.SemaphoreType.DMA((NBUF,)),
    ],
)
def _sc_add(tok_hbm, pos_hbm, out_hbm, pbuf, abuf, psem0, psem1, isem, osem):
    wid = lax.axis_index("s") * NC + lax.axis_index("c")
    t0 = wid * TPW

    pos_cp0 = pltpu.async_copy(
        pos_hbm.at[pl.ds(t0 * D, CH * D)], pbuf.at[pl.ds(0, CH * D)], psem0)
    pos_cp1 = pltpu.async_copy(
        pos_hbm.at[pl.ds((t0 + CH) * D, (TPW - CH) * D)],
        pbuf.at[pl.ds(CH * D, (TPW - CH) * D)], psem1)

    def tok_off(item):
        cc, b = divmod(item, B)
        return b * (T * D) + (t0 + cc * CH) * D

    def start_in(item):
        buf = item % NBUF
        return pltpu.async_copy(
            tok_hbm.at[pl.ds(tok_off(item), CH * D)], abuf.at[buf],
            isem.at[buf])

    in_cp = [None] * NITEMS
    out_cp = [None] * NITEMS
    in_cp[0] = start_in(0)
    in_cp[1] = start_in(1)

    for g in range(NITEMS):
        if g + 2 < NITEMS:
            if g - 2 >= 0:
                out_cp[g - 2].wait()
            in_cp[g + 2] = start_in(g + 2)

        cc = g // B
        if g == 0:
            pos_cp0.wait()
        elif g == B:
            pos_cp1.wait()

        buf = g % NBUF
        in_cp[g].wait()
        ab = abuf.at[buf]
        pbase = cc * CH * D

        @plsc.parallel_loop(0, CH * D, VL, unroll=UNROLL)
        def _body(i, ab=ab, pbase=pbase):
            plsc.addupdate(ab.at[pl.ds(i, VL)], pbuf[pl.ds(pbase + i, VL)])

        out_cp[g] = pltpu.async_copy(
            ab, out_hbm.at[pl.ds(tok_off(g), CH * D)], osem.at[buf])

    out_cp[NITEMS - 2].wait()
    out_cp[NITEMS - 1].wait()


def kernel(encoded_tokens, pos_table):
    tok_flat = encoded_tokens.reshape(B * T * D)
    pos_flat = pos_table.reshape(T * D)
    out_flat = _sc_add(tok_flat, pos_flat)
    return out_flat.reshape(B, T, D)


# speedup vs baseline: 2.9051x; 2.2740x over previous
import functools

import jax
import jax.numpy as jnp
from jax import lax
from jax.experimental import pallas as pl
from jax.experimental.pallas import tpu as pltpu
from jax.experimental.pallas import tpu_sc as plsc

B, T, D = 4, 2048, 1024
NC, NS = 2, 16
NW = NC * NS
TPW = T // NW
CH = 16
NCH = TPW // CH
NITEMS = NCH * B
NBUF = 3
VL = 16
UNROLL = 4

_mesh = plsc.VectorSubcoreMesh(core_axis_name="c", subcore_axis_name="s")


@functools.partial(
    pl.kernel,
    out_type=jax.ShapeDtypeStruct((B, T, D), jnp.float32),
    mesh=_mesh,
    compiler_params=pltpu.CompilerParams(use_tc_tiling_on_sc=True),
    scratch_types=[
        pltpu.VMEM((TPW, D), jnp.float32),
        pltpu.VMEM((NBUF, CH, D), jnp.float32),
        pltpu.SemaphoreType.DMA,
        pltpu.SemaphoreType.DMA,
        pltpu.SemaphoreType.DMA((NBUF,)),
        pltpu.SemaphoreType.DMA((NBUF,)),
    ],
)
def _sc_add(tok_hbm, pos_hbm, out_hbm, pbuf, abuf, psem0, psem1, isem, osem):
    wid = lax.axis_index("s") * NC + lax.axis_index("c")
    t0 = wid * TPW

    pos_cp0 = pltpu.async_copy(
        pos_hbm.at[pl.ds(t0, CH)], pbuf.at[pl.ds(0, CH)], psem0)
    pos_cp1 = pltpu.async_copy(
        pos_hbm.at[pl.ds(t0 + CH, TPW - CH)],
        pbuf.at[pl.ds(CH, TPW - CH)], psem1)

    def coords(item):
        cc, b = divmod(item, B)
        return b, t0 + cc * CH

    def start_in(item):
        buf = item % NBUF
        b, t = coords(item)
        return pltpu.async_copy(
            tok_hbm.at[b, pl.ds(t, CH)], abuf.at[buf], isem.at[buf])

    in_cp = [None] * NITEMS
    out_cp = [None] * NITEMS
    in_cp[0] = start_in(0)

    for g in range(NITEMS):
        if g + 1 < NITEMS:
            if g - 2 >= 0:
                out_cp[g - 2].wait()
            in_cp[g + 1] = start_in(g + 1)

        cc = g // B
        if g == 0:
            pos_cp0.wait()
        elif g == B:
            pos_cp1.wait()

        buf = g % NBUF
        in_cp[g].wait()
        ab = abuf.at[buf]

        @plsc.parallel_loop(0, D, VL, unroll=UNROLL)
        def _body(i, ab=ab, cc=cc):
            for r in range(CH):
                plsc.addupdate(ab.at[r, pl.ds(i, VL)],
                               pbuf[cc * CH + r, pl.ds(i, VL)])

        b, t = coords(g)
        out_cp[g] = pltpu.async_copy(
            ab, out_hbm.at[b, pl.ds(t, CH)], osem.at[buf])

    out_cp[NITEMS - 3].wait()
    out_cp[NITEMS - 2].wait()
    out_cp[NITEMS - 1].wait()


def kernel(encoded_tokens, pos_table):
    return _sc_add(encoded_tokens, pos_table)


# speedup vs baseline: 2.9193x; 1.0049x over previous
import functools

import jax
import jax.numpy as jnp
from jax import lax
from jax.experimental import pallas as pl
from jax.experimental.pallas import tpu as pltpu
from jax.experimental.pallas import tpu_sc as plsc

B, T, D = 4, 2048, 1024
NC, NS = 2, 16
NW = NC * NS
TPW = T // NW
CH = 16
NCH = TPW // CH
NITEMS = NCH * B
NBUF = 3
VL = 16
UNROLL = 4

_mesh = plsc.VectorSubcoreMesh(core_axis_name="c", subcore_axis_name="s")


@functools.partial(
    pl.kernel,
    out_type=jax.ShapeDtypeStruct((B, T, D), jnp.float32),
    mesh=_mesh,
    compiler_params=pltpu.CompilerParams(
        use_tc_tiling_on_sc=True,
        skip_device_barrier=True,
        disable_bounds_checks=True,
        disable_semaphore_checks=True,
    ),
    scratch_types=[
        pltpu.VMEM((TPW, D), jnp.float32),
        pltpu.VMEM((NBUF, CH, D), jnp.float32),
        pltpu.SemaphoreType.DMA,
        pltpu.SemaphoreType.DMA,
        pltpu.SemaphoreType.DMA((NBUF,)),
        pltpu.SemaphoreType.DMA((NBUF,)),
    ],
)
def _sc_add(tok_hbm, pos_hbm, out_hbm, pbuf, abuf, psem0, psem1, isem, osem):
    wid = lax.axis_index("s") * NC + lax.axis_index("c")
    t0 = wid * TPW

    pos_cp0 = pltpu.async_copy(
        pos_hbm.at[pl.ds(t0, CH)], pbuf.at[pl.ds(0, CH)], psem0)
    pos_cp1 = pltpu.async_copy(
        pos_hbm.at[pl.ds(t0 + CH, TPW - CH)],
        pbuf.at[pl.ds(CH, TPW - CH)], psem1)

    def coords(item):
        cc, b = divmod(item, B)
        return b, t0 + cc * CH

    def start_in(item):
        buf = item % NBUF
        b, t = coords(item)
        return pltpu.async_copy(
            tok_hbm.at[b, pl.ds(t, CH)], abuf.at[buf], isem.at[buf])

    in_cp = [None] * NITEMS
    out_cp = [None] * NITEMS
    in_cp[0] = start_in(0)

    for g in range(NITEMS):
        if g + 1 < NITEMS:
            if g - 2 >= 0:
                out_cp[g - 2].wait()
            in_cp[g + 1] = start_in(g + 1)

        cc = g // B
        if g == 0:
            pos_cp0.wait()
        elif g == B:
            pos_cp1.wait()

        buf = g % NBUF
        in_cp[g].wait()
        ab = abuf.at[buf]

        @plsc.parallel_loop(0, D, VL, unroll=UNROLL)
        def _body(i, ab=ab, cc=cc):
            for r in range(CH):
                plsc.addupdate(ab.at[r, pl.ds(i, VL)],
                               pbuf[cc * CH + r, pl.ds(i, VL)])

        b, t = coords(g)
        out_cp[g] = pltpu.async_copy(
            ab, out_hbm.at[b, pl.ds(t, CH)], osem.at[buf])

    out_cp[NITEMS - 3].wait()
    out_cp[NITEMS - 2].wait()
    out_cp[NITEMS - 1].wait()


def kernel(encoded_tokens, pos_table):
    return _sc_add(encoded_tokens, pos_table)


# speedup vs baseline: 3.1225x; 1.0696x over previous
import functools

import jax
import jax.numpy as jnp
from jax import lax
from jax.experimental import pallas as pl
from jax.experimental.pallas import tpu as pltpu
from jax.experimental.pallas import tpu_sc as plsc

B, T, D = 4, 2048, 1024
NC, NS = 2, 16
NW = NC * NS
TPW = T // NW
CH = 8
NCH = TPW // CH
NITEMS = NCH * B
NBUF = 4
VL = 16
UNROLL = 4

_mesh = plsc.VectorSubcoreMesh(core_axis_name="c", subcore_axis_name="s")


@functools.partial(
    pl.kernel,
    out_type=jax.ShapeDtypeStruct((B, T, D), jnp.float32),
    mesh=_mesh,
    compiler_params=pltpu.CompilerParams(
        use_tc_tiling_on_sc=True,
        skip_device_barrier=True,
        disable_bounds_checks=True,
        disable_semaphore_checks=True,
    ),
    scratch_types=[
        pltpu.VMEM((TPW, D), jnp.float32),
        pltpu.VMEM((NBUF, CH, D), jnp.float32),
        pltpu.SemaphoreType.DMA,
        pltpu.SemaphoreType.DMA,
        pltpu.SemaphoreType.DMA((NBUF,)),
        pltpu.SemaphoreType.DMA((NBUF,)),
    ],
)
def _sc_add(tok_hbm, pos_hbm, out_hbm, pbuf, abuf, psem0, psem1, isem, osem):
    wid = lax.axis_index("s") * NC + lax.axis_index("c")
    t0 = wid * TPW

    pos_cp0 = pltpu.async_copy(
        pos_hbm.at[pl.ds(t0, CH)], pbuf.at[pl.ds(0, CH)], psem0)
    pos_cp1 = pltpu.async_copy(
        pos_hbm.at[pl.ds(t0 + CH, TPW - CH)],
        pbuf.at[pl.ds(CH, TPW - CH)], psem1)

    def coords(item):
        cc, b = divmod(item, B)
        return b, t0 + cc * CH

    def start_in(item):
        buf = item % NBUF
        b, t = coords(item)
        return pltpu.async_copy(
            tok_hbm.at[b, pl.ds(t, CH)], abuf.at[buf], isem.at[buf])

    in_cp = [None] * NITEMS
    out_cp = [None] * NITEMS
    in_cp[0] = start_in(0)
    in_cp[1] = start_in(1)

    for g in range(NITEMS):
        if g + 2 < NITEMS:
            if g - 2 >= 0:
                out_cp[g - 2].wait()
            in_cp[g + 2] = start_in(g + 2)

        cc = g // B
        if g == 0:
            pos_cp0.wait()
        elif g == B:
            pos_cp1.wait()

        buf = g % NBUF
        in_cp[g].wait()
        ab = abuf.at[buf]

        @plsc.parallel_loop(0, D, VL, unroll=UNROLL)
        def _body(i, ab=ab, cc=cc):
            for r in range(CH):
                plsc.addupdate(ab.at[r, pl.ds(i, VL)],
                               pbuf[cc * CH + r, pl.ds(i, VL)])

        b, t = coords(g)
        out_cp[g] = pltpu.async_copy(
            ab, out_hbm.at[b, pl.ds(t, CH)], osem.at[buf])

    out_cp[NITEMS - 2].wait()
    out_cp[NITEMS - 1].wait()


def kernel(encoded_tokens, pos_table):
    return _sc_add(encoded_tokens, pos_table)


# speedup vs baseline: 3.2581x; 1.0434x over previous
import functools

import jax
import jax.numpy as jnp
from jax import lax
from jax.experimental import pallas as pl
from jax.experimental.pallas import tpu as pltpu
from jax.experimental.pallas import tpu_sc as plsc

B, T, D = 4, 2048, 1024
NC, NS = 2, 16
NW = NC * NS
TSC = 256
TPW = TSC // NW
CH = 8
NCH = TPW // CH
NITEMS = NCH * B
NBUF = 4
VL = 16
UNROLL = 4

_mesh = plsc.VectorSubcoreMesh(core_axis_name="c", subcore_axis_name="s")


@functools.partial(
    pl.kernel,
    out_type=jax.ShapeDtypeStruct((B, TSC, D), jnp.float32),
    mesh=_mesh,
    compiler_params=pltpu.CompilerParams(
        use_tc_tiling_on_sc=True,
        skip_device_barrier=True,
        disable_bounds_checks=True,
        disable_semaphore_checks=True,
    ),
    scratch_types=[
        pltpu.VMEM((TPW, D), jnp.float32),
        pltpu.VMEM((NBUF, CH, D), jnp.float32),
        pltpu.SemaphoreType.DMA,
        pltpu.SemaphoreType.DMA((NBUF,)),
        pltpu.SemaphoreType.DMA((NBUF,)),
    ],
)
def _sc_add(tok_hbm, pos_hbm, out_hbm, pbuf, abuf, psem, isem, osem):
    wid = lax.axis_index("s") * NC + lax.axis_index("c")
    t0 = wid * TPW

    pos_cp = pltpu.async_copy(pos_hbm.at[pl.ds(t0, TPW)], pbuf, psem)

    def coords(item):
        cc, b = divmod(item, B)
        return b, t0 + cc * CH

    def start_in(item):
        buf = item % NBUF
        b, t = coords(item)
        return pltpu.async_copy(
            tok_hbm.at[b, pl.ds(t, CH)], abuf.at[buf], isem.at[buf])

    in_cp = [None] * NITEMS
    out_cp = [None] * NITEMS
    in_cp[0] = start_in(0)
    if NITEMS > 1:
        in_cp[1] = start_in(1)

    for g in range(NITEMS):
        if g + 2 < NITEMS:
            if g - 2 >= 0:
                out_cp[g - 2].wait()
            in_cp[g + 2] = start_in(g + 2)

        cc = g // B
        if g == 0:
            pos_cp.wait()

        buf = g % NBUF
        in_cp[g].wait()
        ab = abuf.at[buf]

        @plsc.parallel_loop(0, D, VL, unroll=UNROLL)
        def _body(i, ab=ab, cc=cc):
            for r in range(CH):
                plsc.addupdate(ab.at[r, pl.ds(i, VL)],
                               pbuf[cc * CH + r, pl.ds(i, VL)])

        b, t = coords(g)
        out_cp[g] = pltpu.async_copy(
            ab, out_hbm.at[b, pl.ds(t, CH)], osem.at[buf])

    for g in range(max(NITEMS - 2, 0), NITEMS):
        out_cp[g].wait()


def _tc_kernel(tok_ref, pos_ref, out_ref):
    out_ref[...] = tok_ref[...] + pos_ref[...]


TBLK = 256


def _tc_add(encoded_tokens, pos_table):
    ntb = (T - TSC) // TBLK
    return pl.pallas_call(
        _tc_kernel,
        grid=(ntb, B),
        in_specs=[
            pl.BlockSpec((1, TBLK, D), lambda t, b: (b, t + TSC // TBLK, 0)),
            pl.BlockSpec((TBLK, D), lambda t, b: (t + TSC // TBLK, 0)),
        ],
        out_specs=pl.BlockSpec((1, TBLK, D), lambda t, b: (b, t + TSC // TBLK, 0)),
        out_shape=jax.ShapeDtypeStruct((B, T, D), jnp.float32),
    )(encoded_tokens, pos_table)


def kernel(encoded_tokens, pos_table):
    sc_out = _sc_add(encoded_tokens, pos_table)
    tc_out = _tc_add(encoded_tokens, pos_table)
    return lax.dynamic_update_slice(tc_out, sc_out, (0, 0, 0))


# speedup vs baseline: 3.9627x; 1.2163x over previous
import functools

import jax
import jax.numpy as jnp
from jax import lax
from jax.experimental import pallas as pl
from jax.experimental.pallas import tpu as pltpu
from jax.experimental.pallas import tpu_sc as plsc

B, T, D = 4, 2048, 1024
NC, NS = 2, 16
NW = NC * NS
TSC = 256
TPW = TSC // NW
CH = 8
NCH = TPW // CH
NITEMS = NCH * B
NBUF = 4
VL = 16
UNROLL = 4

_mesh = plsc.VectorSubcoreMesh(core_axis_name="c", subcore_axis_name="s")


@functools.partial(
    pl.kernel,
    out_type=jax.ShapeDtypeStruct((B, TSC, D), jnp.float32),
    mesh=_mesh,
    compiler_params=pltpu.CompilerParams(
        use_tc_tiling_on_sc=True,
        skip_device_barrier=True,
        disable_bounds_checks=True,
        disable_semaphore_checks=True,
    ),
    scratch_types=[
        pltpu.VMEM((TPW, D), jnp.float32),
        pltpu.VMEM((NBUF, CH, D), jnp.float32),
        pltpu.SemaphoreType.DMA,
        pltpu.SemaphoreType.DMA((NBUF,)),
        pltpu.SemaphoreType.DMA((NBUF,)),
    ],
)
def _sc_add(tok_hbm, pos_hbm, out_hbm, pbuf, abuf, psem, isem, osem):
    wid = lax.axis_index("s") * NC + lax.axis_index("c")
    t0 = wid * TPW

    pos_cp = pltpu.async_copy(pos_hbm.at[pl.ds(t0, TPW)], pbuf, psem)

    def coords(item):
        cc, b = divmod(item, B)
        return b, t0 + cc * CH

    def start_in(item):
        buf = item % NBUF
        b, t = coords(item)
        return pltpu.async_copy(
            tok_hbm.at[b, pl.ds(t, CH)], abuf.at[buf], isem.at[buf])

    in_cp = [None] * NITEMS
    out_cp = [None] * NITEMS
    in_cp[0] = start_in(0)
    if NITEMS > 1:
        in_cp[1] = start_in(1)

    for g in range(NITEMS):
        if g + 2 < NITEMS:
            if g - 2 >= 0:
                out_cp[g - 2].wait()
            in_cp[g + 2] = start_in(g + 2)

        cc = g // B
        if g == 0:
            pos_cp.wait()

        buf = g % NBUF
        in_cp[g].wait()
        ab = abuf.at[buf]

        @plsc.parallel_loop(0, D, VL, unroll=UNROLL)
        def _body(i, ab=ab, cc=cc):
            for r in range(CH):
                plsc.addupdate(ab.at[r, pl.ds(i, VL)],
                               pbuf[cc * CH + r, pl.ds(i, VL)])

        b, t = coords(g)
        out_cp[g] = pltpu.async_copy(
            ab, out_hbm.at[b, pl.ds(t, CH)], osem.at[buf])

    for g in range(max(NITEMS - 2, 0), NITEMS):
        out_cp[g].wait()


def _tc_kernel(tok_ref, pos_ref, out_ref):
    out_ref[...] = tok_ref[...] + pos_ref[...]


TBLK = 512


def _tc_add(encoded_tokens, pos_table):
    ntb = (T - TSC) // TBLK
    return pl.pallas_call(
        _tc_kernel,
        grid=(ntb, B),
        in_specs=[
            pl.BlockSpec((1, TBLK, D), lambda t, b: (b, t + TSC // TBLK, 0)),
            pl.BlockSpec((TBLK, D), lambda t, b: (t + TSC // TBLK, 0)),
        ],
        out_specs=pl.BlockSpec((1, TBLK, D), lambda t, b: (b, t + TSC // TBLK, 0)),
        out_shape=jax.ShapeDtypeStruct((B, T, D), jnp.float32),
    )(encoded_tokens, pos_table)


def kernel(encoded_tokens, pos_table):
    sc_out = _sc_add(encoded_tokens, pos_table)
    tc_out = _tc_add(encoded_tokens, pos_table)
    return lax.dynamic_update_slice(tc_out, sc_out, (0, 0, 0))
